# Initial kernel scaffold; baseline (speedup 1.0000x reference)
import jax
import jax.numpy as jnp
from jax.experimental import pallas as pl


def kernel(node_in, edge_index, edge_in, W1, We1, al1, ar1, ae1, b1, W2, We2, al2, ar2, ae2, b2, W3, We3, al3, ar3, ae3, b3):
    raise NotImplementedError("write your pallas kernel here")


# speedup vs baseline: 1.7919x; 1.7919x over previous
import functools

import jax
import jax.numpy as jnp
from jax.experimental import pallas as pl
from jax.experimental.pallas import tpu as pltpu

_N = 10000
_E = 320000
_D = 128
_DE = 16
_H = 128


def _dense_in(x, W, al, ar):
    def body(x_ref, w_ref, a_ref, ft_ref, el_ref, er_ref):
        ft = jnp.dot(x_ref[...], w_ref[...], preferred_element_type=jnp.float32)
        ft_ref[...] = ft
        el_ref[...] = jnp.dot(ft, a_ref[...][:, 0:1])
        er_ref[...] = jnp.dot(ft, a_ref[...][:, 1:2])

    a = jnp.stack([al, ar], axis=1)
    ft, el, er = pl.pallas_call(
        body,
        out_shape=(
            jax.ShapeDtypeStruct((_N, _H), jnp.float32),
            jax.ShapeDtypeStruct((_N, 1), jnp.float32),
            jax.ShapeDtypeStruct((_N, 1), jnp.float32),
        ),
    )(x, W, a)
    return ft, el[:, 0], er[:, 0]


def _eterm(edge_in, We, ae):
    def body(e_ref, w_ref, ae_ref, out_ref):
        w = jnp.dot(w_ref[...], ae_ref[...])
        wfull = jnp.concatenate([w] * 8, axis=0)
        ic = jax.lax.broadcasted_iota(jnp.int32, (8 * _DE, 8), 0)
        ik = jax.lax.broadcasted_iota(jnp.int32, (8 * _DE, 8), 1)
        wtile = jnp.where((ic // _DE) == ik, wfull, 0.0)
        out_ref[...] = jnp.dot(e_ref[...], wtile,
                               preferred_element_type=jnp.float32)

    out = pl.pallas_call(
        body,
        out_shape=jax.ShapeDtypeStruct((_E // 8, 8), jnp.float32),
    )(edge_in.reshape(_E // 8, 8 * _DE), We, ae.reshape(_H, 1))
    return out.reshape(_E)


def _combine(acc128, acc32, We, b, apply_tanh):
    def body(a128_ref, a32_ref, we_ref, b_ref, h_ref):
        num = a128_ref[0] + a128_ref[1]
        s32 = a32_ref[0] + a32_ref[1]
        se = s32[:, 0:_DE]
        den = s32[:, _DE:_DE + 1]
        h = (num + jnp.dot(se, we_ref[...], preferred_element_type=jnp.float32))
        h = h / jnp.maximum(den, 1e-9) + b_ref[...]
        if apply_tanh:
            h = jnp.tanh(h)
        h_ref[...] = h

    return pl.pallas_call(
        body,
        out_shape=jax.ShapeDtypeStruct((_N, _H), jnp.float32),
    )(acc128, acc32, We, b.reshape(1, _H))


def _edge_pass(ft, el, er, eterm, edge_in, src, dst):
    e = el[src] + er[dst] + eterm
    e = jnp.where(e > 0, e, 0.2 * e)
    ex = jnp.exp(e)
    num = jax.ops.segment_sum(ex[:, None] * ft[src], dst, num_segments=_N)
    s16 = jax.ops.segment_sum(ex[:, None] * edge_in, dst, num_segments=_N)
    den = jax.ops.segment_sum(ex, dst, num_segments=_N)
    z = jnp.zeros((_N, 32 - _DE - 1), jnp.float32)
    acc32 = jnp.concatenate([s16, den[:, None], z], axis=1)
    zeros128 = jnp.zeros((_N, _H), jnp.float32)
    zeros32 = jnp.zeros((_N, 32), jnp.float32)
    return (jnp.stack([num, zeros128]), jnp.stack([acc32, zeros32]))


def _layer(x, edge_in, src, dst, W, We, al, ar, eterm, b, apply_tanh):
    ft, el, er = _dense_in(x, W, al, ar)
    acc128, acc32 = _edge_pass(ft, el, er, eterm, edge_in, src, dst)
    return _combine(acc128, acc32, We, b, apply_tanh)


def kernel(node_in, edge_index, edge_in,
           W1, We1, al1, ar1, ae1, b1,
           W2, We2, al2, ar2, ae2, b2,
           W3, We3, al3, ar3, ae3, b3):
    src = edge_index[0]
    dst = edge_index[1]
    et1 = _eterm(edge_in, We1, ae1)
    et2 = _eterm(edge_in, We2, ae2)
    et3 = _eterm(edge_in, We3, ae3)
    h = _layer(node_in, edge_in, src, dst, W1, We1, al1, ar1, et1, b1, True)
    h = _layer(h, edge_in, src, dst, W2, We2, al2, ar2, et2, b2, True)
    h = _layer(h, edge_in, src, dst, W2, We2, al2, ar2, et2, b2, True)
    h = _layer(h, edge_in, src, dst, W3, We3, al3, ar3, et3, b3, False)
    return h


# speedup vs baseline: 13.9352x; 7.7769x over previous
import dataclasses
import functools

import jax
import jax.numpy as jnp
from jax import lax
from jax.experimental import pallas as pl
from jax.experimental.pallas import tpu as pltpu
from jax.experimental.pallas import tpu_sc as plsc

_N = 10000
_E = 320000
_D = 128
_DE = 16
_H = 128

_NC = 2
_NS = 16
_NL = 16
_NW = _NC * _NS
_CH = 128
_NCHUNK = _E // _CH
_NPAD = 10112
_RPT = _NPAD // _NS


def _dense_in(x, W, al, ar):
    def body(x_ref, w_ref, a_ref, ft_ref, el_ref, er_ref):
        ft = jnp.dot(x_ref[...], w_ref[...], preferred_element_type=jnp.float32)
        ft_ref[...] = ft
        el_ref[...] = jnp.dot(ft, a_ref[...][:, 0:1])
        er_ref[...] = jnp.dot(ft, a_ref[...][:, 1:2])

    a = jnp.stack([al, ar], axis=1)
    ft, el, er = pl.pallas_call(
        body,
        out_shape=(
            jax.ShapeDtypeStruct((_N, _H), jnp.float32),
            jax.ShapeDtypeStruct((_N, 1), jnp.float32),
            jax.ShapeDtypeStruct((_N, 1), jnp.float32),
        ),
    )(x, W, a)
    return ft, el[:, 0], er[:, 0]


def _eterm(edge_in, We, ae):
    def body(e_ref, w_ref, ae_ref, out_ref):
        w = jnp.dot(w_ref[...], ae_ref[...])
        wfull = jnp.concatenate([w] * 8, axis=0)
        ic = jax.lax.broadcasted_iota(jnp.int32, (8 * _DE, 8), 0)
        ik = jax.lax.broadcasted_iota(jnp.int32, (8 * _DE, 8), 1)
        wtile = jnp.where((ic // _DE) == ik, wfull, 0.0)
        out_ref[...] = jnp.dot(e_ref[...], wtile,
                               preferred_element_type=jnp.float32)

    out = pl.pallas_call(
        body,
        out_shape=jax.ShapeDtypeStruct((_E // 8, 8), jnp.float32),
    )(edge_in.reshape(_E // 8, 8 * _DE), We, ae.reshape(_H, 1))
    return out.reshape(_E)


def _combine(acc128, acc32, We, b, apply_tanh):
    def body(a128_ref, a32_ref, we_ref, b_ref, h_ref):
        num = a128_ref[0, :_N] + a128_ref[1, :_N]
        s32 = a32_ref[0, :_N] + a32_ref[1, :_N]
        se = s32[:, 0:_DE]
        den = s32[:, _DE:_DE + 1]
        h = (num + jnp.dot(se, we_ref[...], preferred_element_type=jnp.float32))
        h = h / jnp.maximum(den, 1e-9) + b_ref[...]
        if apply_tanh:
            h = jnp.tanh(h)
        h_ref[...] = h

    return pl.pallas_call(
        body,
        out_shape=jax.ShapeDtypeStruct((_N, _H), jnp.float32),
    )(acc128, acc32, We, b.reshape(1, _H))


def _edge_pass(ft, el, er, eterm, ein2, src, dst):
    mesh = plsc.VectorSubcoreMesh(core_axis_name="c", subcore_axis_name="s")
    cp = pltpu.CompilerParams(use_tc_tiling_on_sc=False)
    if "needs_layout_passes" in pltpu.CompilerParams.__dataclass_fields__:
        cp = dataclasses.replace(cp, needs_layout_passes=False)

    @functools.partial(
        pl.kernel,
        out_type=(jax.ShapeDtypeStruct((_NC, _NPAD, _H), jnp.float32),
                  jax.ShapeDtypeStruct((_NC, _NPAD, 32), jnp.float32)),
        mesh=mesh,
        compiler_params=cp,
        scratch_types=[
            pltpu.VMEM_SHARED((_NPAD, _H), jnp.float32),
            pltpu.VMEM_SHARED((_NPAD, 32), jnp.float32),
            pltpu.VMEM((_CH,), jnp.float32),
            pltpu.VMEM((_CH,), jnp.float32),
            pltpu.VMEM((_CH,), jnp.int32),
            pltpu.VMEM((_CH,), jnp.int32),
            pltpu.VMEM((_CH,), jnp.float32),
            pltpu.VMEM((_CH // 8, _H), jnp.float32),
            pltpu.VMEM((_CH, _H), jnp.float32),
            pltpu.VMEM((_CH, 32), jnp.float32),
            pltpu.VMEM((_CH,), jnp.float32),
            pltpu.SemaphoreType.DMA,
        ],
    )
    def k(ft_hbm, el_hbm, er_hbm, et_hbm, ein_hbm, src_hbm, dst_hbm,
          out128_hbm, out32_hbm,
          acc128, acc32, elg_v, erg_v, src_v, dst_v, et_v, ein_v, ftb, st2,
          exb, sem):
        c = lax.axis_index("c")
        s = lax.axis_index("s")
        wid = s * _NC + c
        zv = jnp.zeros((_NL,), jnp.float32)

        @pl.loop(0, _CH)
        def _zrow(i):
            for kk in range(_H // _NL):
                ftb[i, pl.ds(kk * _NL, _NL)] = zv
            st2[i, pl.ds(0, _NL)] = zv
            st2[i, pl.ds(_NL, _NL)] = zv

        r0 = s * _RPT
        for j in range(0, _RPT, _CH):
            nr = min(_CH, _RPT - j)
            pltpu.sync_copy(ftb.at[pl.ds(0, nr)], acc128.at[pl.ds(r0 + j, nr)])
            pltpu.sync_copy(st2.at[pl.ds(0, nr)], acc32.at[pl.ds(r0 + j, nr)])
        plsc.subcore_barrier()

        iota = lax.iota(jnp.int32, _NL)
        one0 = jnp.where(iota == 0, jnp.float32(1), jnp.float32(0))

        @pl.loop(wid, _NCHUNK, step=_NW)
        def _chunk(g):
            base = g * _CH
            pltpu.sync_copy(src_hbm.at[pl.ds(base, _CH)], src_v)
            pltpu.sync_copy(dst_hbm.at[pl.ds(base, _CH)], dst_v)
            pltpu.sync_copy(et_hbm.at[pl.ds(base, _CH)], et_v)
            pltpu.sync_copy(ein_hbm.at[pl.ds(g * (_CH // 8), _CH // 8)], ein_v)
            pltpu.sync_copy(el_hbm.at[src_v], elg_v)
            pltpu.sync_copy(er_hbm.at[dst_v], erg_v)
            pltpu.async_copy(ft_hbm.at[src_v], ftb, sem).wait()

            @pl.loop(0, _CH, step=_NL)
            def _ex16(i):
                e = elg_v[pl.ds(i, _NL)] + erg_v[pl.ds(i, _NL)]
                e = e + et_v[pl.ds(i, _NL)]
                e = jnp.maximum(e, 0.2 * e)
                exb[pl.ds(i, _NL)] = jnp.exp(e)

            @pl.loop(0, _CH)
            def _edge(i):
                bex = plsc.load_gather(exb, [jnp.broadcast_to(i, (_NL,))])
                for kk in range(_H // _NL):
                    sl = pl.ds(kk * _NL, _NL)
                    ftb[i, sl] = ftb[i, sl] * bex
                st2[i, pl.ds(0, _NL)] = ein_v[i // 8, pl.ds((i % 8) * _DE, _DE)] * bex
                st2[i, pl.ds(_NL, _NL)] = bex * one0

            pltpu.sync_copy(ftb, acc128.at[dst_v], add=True)
            pltpu.sync_copy(st2, acc32.at[dst_v], add=True)

        plsc.subcore_barrier()
        pltpu.sync_copy(acc128.at[pl.ds(r0, _RPT)],
                        out128_hbm.at[c, pl.ds(r0, _RPT)])
        pltpu.sync_copy(acc32.at[pl.ds(r0, _RPT)],
                        out32_hbm.at[c, pl.ds(r0, _RPT)])

    return k(ft, el, er, eterm, ein2, src, dst)


def _layer(x, ein2, src, dst, W, We, al, ar, eterm, b, apply_tanh):
    ft, el, er = _dense_in(x, W, al, ar)
    acc128, acc32 = _edge_pass(ft, el, er, eterm, ein2, src, dst)
    return _combine(acc128, acc32, We, b, apply_tanh)


def kernel(node_in, edge_index, edge_in,
           W1, We1, al1, ar1, ae1, b1,
           W2, We2, al2, ar2, ae2, b2,
           W3, We3, al3, ar3, ae3, b3):
    src = edge_index[0].astype(jnp.int32)
    dst = edge_index[1].astype(jnp.int32)
    ein2 = edge_in.reshape(_E // 8, 8 * _DE)
    et1 = _eterm(edge_in, We1, ae1)
    et2 = _eterm(edge_in, We2, ae2)
    et3 = _eterm(edge_in, We3, ae3)
    h = _layer(node_in, ein2, src, dst, W1, We1, al1, ar1, et1, b1, True)
    h = _layer(h, ein2, src, dst, W2, We2, al2, ar2, et2, b2, True)
    h = _layer(h, ein2, src, dst, W2, We2, al2, ar2, et2, b2, True)
    h = _layer(h, ein2, src, dst, W3, We3, al3, ar3, et3, b3, False)
    return h


# speedup vs baseline: 19.3695x; 1.3900x over previous
import dataclasses
import functools

import jax
import jax.numpy as jnp
from jax import lax
from jax.experimental import pallas as pl
from jax.experimental.pallas import tpu as pltpu
from jax.experimental.pallas import tpu_sc as plsc

_N = 10000
_E = 320000
_D = 128
_DE = 16
_H = 128

_NC = 2
_NS = 16
_NL = 16
_NW = _NC * _NS
_CH = 128
_NCHUNK = _E // _CH
_NPAD = 10112
_RPT = _NPAD // _NS


def _dense_in(x, W, al, ar):
    def body(x_ref, w_ref, a_ref, ft_ref, el_ref, er_ref):
        ft = jnp.dot(x_ref[...], w_ref[...], preferred_element_type=jnp.float32)
        ft_ref[...] = ft
        el_ref[...] = jnp.dot(ft, a_ref[...][:, 0:1])
        er_ref[...] = jnp.dot(ft, a_ref[...][:, 1:2])

    a = jnp.stack([al, ar], axis=1)
    ft, el, er = pl.pallas_call(
        body,
        out_shape=(
            jax.ShapeDtypeStruct((_N, _H), jnp.float32),
            jax.ShapeDtypeStruct((_N, 1), jnp.float32),
            jax.ShapeDtypeStruct((_N, 1), jnp.float32),
        ),
    )(x, W, a)
    return ft, el[:, 0], er[:, 0]


def _eterm(edge_in, We, ae):
    def body(e_ref, w_ref, ae_ref, out_ref):
        w = jnp.dot(w_ref[...], ae_ref[...])
        wfull = jnp.concatenate([w] * 8, axis=0)
        ic = jax.lax.broadcasted_iota(jnp.int32, (8 * _DE, 8), 0)
        ik = jax.lax.broadcasted_iota(jnp.int32, (8 * _DE, 8), 1)
        wtile = jnp.where((ic // _DE) == ik, wfull, 0.0)
        out_ref[...] = jnp.dot(e_ref[...], wtile,
                               preferred_element_type=jnp.float32)

    out = pl.pallas_call(
        body,
        out_shape=jax.ShapeDtypeStruct((_E // 8, 8), jnp.float32),
    )(edge_in.reshape(_E // 8, 8 * _DE), We, ae.reshape(_H, 1))
    return out.reshape(_E)


def _combine(acc128, acc32, We, b, apply_tanh):
    def body(a128_ref, a32_ref, we_ref, b_ref, h_ref):
        num = a128_ref[0, :_N] + a128_ref[1, :_N]
        s32 = a32_ref[0, :_N] + a32_ref[1, :_N]
        se = s32[:, 0:_DE]
        den = s32[:, _DE:_DE + 1]
        h = (num + jnp.dot(se, we_ref[...], preferred_element_type=jnp.float32))
        h = h / jnp.maximum(den, 1e-9) + b_ref[...]
        if apply_tanh:
            h = jnp.tanh(h)
        h_ref[...] = h

    return pl.pallas_call(
        body,
        out_shape=jax.ShapeDtypeStruct((_N, _H), jnp.float32),
    )(acc128, acc32, We, b.reshape(1, _H))


def _edge_pass(ft, el, er, eterm, ein2, src, dst):
    mesh = plsc.VectorSubcoreMesh(core_axis_name="c", subcore_axis_name="s")
    cp = pltpu.CompilerParams(use_tc_tiling_on_sc=False)
    if "needs_layout_passes" in pltpu.CompilerParams.__dataclass_fields__:
        cp = dataclasses.replace(cp, needs_layout_passes=False)

    @functools.partial(
        pl.kernel,
        out_type=(jax.ShapeDtypeStruct((_NC, _NPAD, _H), jnp.float32),
                  jax.ShapeDtypeStruct((_NC, _NPAD, 32), jnp.float32)),
        mesh=mesh,
        compiler_params=cp,
        scratch_types=[
            pltpu.VMEM_SHARED((_NPAD, _H), jnp.float32),
            pltpu.VMEM_SHARED((_NPAD, 32), jnp.float32),
            pltpu.VMEM((_CH,), jnp.float32),
            pltpu.VMEM((_CH,), jnp.float32),
            pltpu.VMEM((_CH,), jnp.int32),
            pltpu.VMEM((_CH,), jnp.int32),
            pltpu.VMEM((_CH,), jnp.float32),
            pltpu.VMEM((_CH // 8, _H), jnp.float32),
            pltpu.VMEM((_CH, _H), jnp.float32),
            pltpu.VMEM((_CH, 32), jnp.float32),
            pltpu.VMEM((_CH,), jnp.float32),
            pltpu.SemaphoreType.DMA,
        ],
    )
    def k(ft_hbm, el_hbm, er_hbm, et_hbm, ein_hbm, src_hbm, dst_hbm,
          out128_hbm, out32_hbm,
          acc128, acc32, elg_v, erg_v, src_v, dst_v, et_v, ein_v, ftb, st2,
          exb, sem):
        c = lax.axis_index("c")
        s = lax.axis_index("s")
        wid = s * _NC + c
        zv = jnp.zeros((_NL,), jnp.float32)

        @pl.loop(0, _CH)
        def _zrow(i):
            for kk in range(_H // _NL):
                ftb[i, pl.ds(kk * _NL, _NL)] = zv
            st2[i, pl.ds(0, _NL)] = zv
            st2[i, pl.ds(_NL, _NL)] = zv

        r0 = s * _RPT
        for j in range(0, _RPT, _CH):
            nr = min(_CH, _RPT - j)
            pltpu.sync_copy(ftb.at[pl.ds(0, nr)], acc128.at[pl.ds(r0 + j, nr)])
            pltpu.sync_copy(st2.at[pl.ds(0, nr)], acc32.at[pl.ds(r0 + j, nr)])
        plsc.subcore_barrier()

        iota = lax.iota(jnp.int32, _NL)
        one0 = jnp.where(iota == 0, jnp.float32(1), jnp.float32(0))

        @pl.loop(wid, _NCHUNK, step=_NW)
        def _chunk(g):
            base = g * _CH
            d1 = pltpu.async_copy(src_hbm.at[pl.ds(base, _CH)], src_v, sem)
            d2 = pltpu.async_copy(dst_hbm.at[pl.ds(base, _CH)], dst_v, sem)
            d3 = pltpu.async_copy(et_hbm.at[pl.ds(base, _CH)], et_v, sem)
            d4 = pltpu.async_copy(
                ein_hbm.at[pl.ds(g * (_CH // 8), _CH // 8)], ein_v, sem)
            d1.wait(); d2.wait(); d3.wait(); d4.wait()
            g1 = pltpu.async_copy(el_hbm.at[src_v], elg_v, sem)
            g2 = pltpu.async_copy(er_hbm.at[dst_v], erg_v, sem)
            g3 = pltpu.async_copy(ft_hbm.at[src_v], ftb, sem)
            g1.wait(); g2.wait(); g3.wait()

            @pl.loop(0, _CH, step=_NL)
            def _ex16(i):
                e = elg_v[pl.ds(i, _NL)] + erg_v[pl.ds(i, _NL)]
                e = e + et_v[pl.ds(i, _NL)]
                e = jnp.maximum(e, 0.2 * e)
                exb[pl.ds(i, _NL)] = jnp.exp(e)

            @pl.loop(0, _CH)
            def _edge(i):
                bex = plsc.load_gather(exb, [jnp.broadcast_to(i, (_NL,))])
                for kk in range(_H // _NL):
                    sl = pl.ds(kk * _NL, _NL)
                    ftb[i, sl] = ftb[i, sl] * bex
                st2[i, pl.ds(0, _NL)] = ein_v[i // 8, pl.ds((i % 8) * _DE, _DE)] * bex
                st2[i, pl.ds(_NL, _NL)] = bex * one0

            s1 = pltpu.async_copy(ftb, acc128.at[dst_v], sem, add=True)
            s2 = pltpu.async_copy(st2, acc32.at[dst_v], sem, add=True)
            s1.wait(); s2.wait()

        plsc.subcore_barrier()
        pltpu.sync_copy(acc128.at[pl.ds(r0, _RPT)],
                        out128_hbm.at[c, pl.ds(r0, _RPT)])
        pltpu.sync_copy(acc32.at[pl.ds(r0, _RPT)],
                        out32_hbm.at[c, pl.ds(r0, _RPT)])

    return k(ft, el, er, eterm, ein2, src, dst)


def _layer(x, ein2, src, dst, W, We, al, ar, eterm, b, apply_tanh):
    ft, el, er = _dense_in(x, W, al, ar)
    acc128, acc32 = _edge_pass(ft, el, er, eterm, ein2, src, dst)
    return _combine(acc128, acc32, We, b, apply_tanh)


def kernel(node_in, edge_index, edge_in,
           W1, We1, al1, ar1, ae1, b1,
           W2, We2, al2, ar2, ae2, b2,
           W3, We3, al3, ar3, ae3, b3):
    src = edge_index[0].astype(jnp.int32)
    dst = edge_index[1].astype(jnp.int32)
    ein2 = edge_in.reshape(_E // 8, 8 * _DE)
    et1 = _eterm(edge_in, We1, ae1)
    et2 = _eterm(edge_in, We2, ae2)
    et3 = _eterm(edge_in, We3, ae3)
    h = _layer(node_in, ein2, src, dst, W1, We1, al1, ar1, et1, b1, True)
    h = _layer(h, ein2, src, dst, W2, We2, al2, ar2, et2, b2, True)
    h = _layer(h, ein2, src, dst, W2, We2, al2, ar2, et2, b2, True)
    h = _layer(h, ein2, src, dst, W3, We3, al3, ar3, et3, b3, False)
    return h
